# baseline (speedup 1.0000x reference)
import functools

import jax
import jax.numpy as jnp
from jax import lax
from jax.experimental import pallas as pl
from jax.experimental.pallas import tpu as pltpu
from jax.experimental.pallas import tpu_sc as plsc

NVARS = 25000
NCONS = 20000
NCUTS = 5000
N = NVARS + NCONS + NCUTS
EC = 1280000
EK = 320000
E = EC + EK
VF = 13
CF = 5
KF = 13
OC = 32

NCORES = 2
NSUB = 16
NW = NCORES * NSUB
CHUNK = 128

NDST_V = 25088
NDST_C = 5008
TCH_C = 314
TCH_K = 80
PE_C = NW * TCH_C * CHUNK
PE_K = NW * TCH_K * CHUNK

_GDN = lax.GatherDimensionNumbers(offset_dims=(), collapsed_slice_dims=(0,),
                                  start_index_map=(0,))


def _lane_bcast(vec16, lane):
    idx = jnp.full((16, 1), lane, jnp.int32)
    return lax.gather(vec16, idx, _GDN, (1,),
                      mode=lax.GatherScatterMode.PROMISE_IN_BOUNDS)


def _cdiv(a, b):
    return (a + b - 1) // b


def _make_edge_aggr(tchunks, ndst_pad):
    rpt = ndst_pad // NSUB
    mesh = plsc.VectorSubcoreMesh(core_axis_name="c", subcore_axis_name="s")

    @functools.partial(
        pl.kernel,
        mesh=mesh,
        compiler_params=pltpu.CompilerParams(use_tc_tiling_on_sc=False,
                                             needs_layout_passes=False),
        out_type=[
            jax.ShapeDtypeStruct((NCORES * ndst_pad, 32), jnp.float32),
            jax.ShapeDtypeStruct((NW, ndst_pad), jnp.float32),
        ],
        scratch_types=[
            pltpu.VMEM((3, 128), jnp.int32),
            pltpu.VMEM((3, 128), jnp.int32),
            pltpu.VMEM((CHUNK, 16), jnp.float32),
            pltpu.VMEM((CHUNK, 16), jnp.float32),
            pltpu.VMEM((128,), jnp.int32),
            pltpu.VMEM((128,), jnp.int32),
            pltpu.VMEM((CHUNK, 32), jnp.float32),
            pltpu.VMEM((CHUNK, 32), jnp.float32),
            pltpu.VMEM((CHUNK, 32), jnp.float32),
            pltpu.VMEM((CHUNK, 32), jnp.float32),
            pltpu.VMEM((CHUNK, 32), jnp.float32),
            pltpu.VMEM((CHUNK, 32), jnp.float32),
            pltpu.VMEM((32,), jnp.float32),
            pltpu.VMEM((ndst_pad,), jnp.float32),
            pltpu.VMEM_SHARED((ndst_pad, 32), jnp.float32),
            pltpu.SemaphoreType.DMA,
            pltpu.SemaphoreType.DMA,
            pltpu.SemaphoreType.DMA,
            pltpu.SemaphoreType.DMA,
            pltpu.SemaphoreType.DMA,
            pltpu.SemaphoreType.DMA,
            pltpu.SemaphoreType.DMA,
            pltpu.SemaphoreType.DMA,
        ],
    )
    def k(t_hbm, ie_hbm, il_hbm, z32_hbm, zc_hbm, w_hbm, out_acc, out_cnt,
          ie0, ie1, il0, il1, ds0, ds1, ra0, ra1, rb0, rb1, ov0, ov1, w_v,
          cnt_v, acc_s, se0, se1, sl0, sl1, sg0, sg1, ss0, ss1):
        c = lax.axis_index("c")
        s = lax.axis_index("s")
        wid = c * NSUB + s
        ie = (ie0, ie1)
        il = (il0, il1)
        ds = (ds0, ds1)
        ra = (ra0, ra1)
        rb = (rb0, rb1)
        ov = (ov0, ov1)
        sem_e = (se0, se1)
        sem_l = (sl0, sl1)
        sem_g = (sg0, sg1)
        sem_s = (ss0, ss1)

        r0 = s * rpt
        pltpu.sync_copy(z32_hbm.at[pl.ds(r0, rpt)], acc_s.at[pl.ds(r0, rpt)])
        pltpu.sync_copy(zc_hbm, cnt_v)
        pltpu.sync_copy(w_hbm, w_v)
        w_lo = w_v[pl.ds(0, 16)]
        w_hi = w_v[pl.ds(16, 16)]
        plsc.subcore_barrier()

        base = wid * tchunks

        def start_e(i, slot):
            return pltpu.async_copy(ie_hbm.at[base + i], ie[slot], sem_e[slot])

        def start_l(i, slot):
            return pltpu.async_copy(il_hbm.at[base + i], il[slot], sem_l[slot])

        def start_gathers(i, slot):
            del i
            pltpu.async_copy(t_hbm.at[ie[slot].at[0]], ra[slot], sem_g[slot])
            pltpu.async_copy(t_hbm.at[ie[slot].at[1]], rb[slot], sem_g[slot])

        def wait_gathers(slot):
            pltpu.make_async_copy(t_hbm.at[ie[slot].at[0]], ra[slot],
                                  sem_g[slot]).wait()
            pltpu.make_async_copy(t_hbm.at[ie[slot].at[1]], rb[slot],
                                  sem_g[slot]).wait()

        start_e(0, 0)
        start_l(0, 0)
        start_e(1, 1)
        start_l(1, 1)
        pltpu.make_async_copy(ie_hbm.at[0], ie[0], sem_e[0]).wait()
        start_gathers(0, 0)

        def iteration(i, slot):
            oslot = 1 - slot
            @pl.when(i + 1 < tchunks)
            def _():
                pltpu.make_async_copy(ie_hbm.at[0], ie[oslot],
                                      sem_e[oslot]).wait()
                start_gathers(i + 1, oslot)

            wait_gathers(slot)
            pltpu.make_async_copy(il_hbm.at[0], il[slot], sem_l[slot]).wait()

            @pl.when(i >= 2)
            def _():
                pltpu.make_async_copy(ov[slot], acc_s.at[ds[slot]],
                                      sem_s[slot]).wait()

            rav = ra[slot]
            rbv = rb[slot]
            ovv = ov[slot]
            ilv = il[slot]

            one16 = jnp.ones((16,), jnp.float32)
            for t in range(CHUNK // 16):
                ds16 = ie[slot][2, pl.ds(t * 16, 16)]
                ds[slot][pl.ds(t * 16, 16)] = ds16
                plsc.addupdate_scatter(cnt_v, [ds16], one16)

            def group(g, carry):
                gbase = g * 16
                for u in range(16):
                    e = gbase + u
                    av = ilv[e]
                    v0 = rav[e, pl.ds(0, 16)] + rbv[e, pl.ds(0, 16)] + av * w_lo
                    ovv[e, pl.ds(0, 16)] = jnp.maximum(v0, 0.0)
                    v1 = (rav[e, pl.ds(16, 16)] + rbv[e, pl.ds(16, 16)]
                          + av * w_hi)
                    ovv[e, pl.ds(16, 16)] = jnp.maximum(v1, 0.0)
                return carry

            lax.fori_loop(0, CHUNK // 16, group, 0, unroll=False)

            pltpu.async_copy(ovv, acc_s.at[ds[slot]], sem_s[slot], add=True)

            @pl.when(i + 2 < tchunks)
            def _():
                start_e(i + 2, slot)
                start_l(i + 2, slot)

        def pair_body(p, carry):
            iteration(2 * p, 0)
            iteration(2 * p + 1, 1)
            return carry

        lax.fori_loop(0, tchunks // 2, pair_body, 0, unroll=False)
        pltpu.make_async_copy(ov[0], acc_s.at[ds[0]], sem_s[0]).wait()
        pltpu.make_async_copy(ov[1], acc_s.at[ds[1]], sem_s[1]).wait()
        plsc.subcore_barrier()

        o0 = c * ndst_pad + r0
        pltpu.sync_copy(acc_s.at[pl.ds(r0, rpt)], out_acc.at[pl.ds(o0, rpt)])
        pltpu.sync_copy(cnt_v, out_cnt.at[wid])

    return k


_edge_aggr_g = _make_edge_aggr(TCH_C, NDST_V)
_edge_aggr_h = _make_edge_aggr(TCH_K, NDST_V)
_edge_aggr_d = _make_edge_aggr(TCH_K, NDST_C)


def _proj_body(x_ref, wgc_ref, wgr_ref, bg_ref, whc_ref, whr_ref, bh_ref,
               tg_ref, th_ref):
    xb = x_ref[...]
    row0 = pl.program_id(0) * xb.shape[0]
    gr = row0 + lax.broadcasted_iota(jnp.int32, (xb.shape[0], 1), 0)
    is_var = gr < NVARS
    tg_col = jnp.dot(xb, wgc_ref[...], preferred_element_type=jnp.float32)
    tg_row = jnp.dot(xb[:, :CF], wgr_ref[...],
                     preferred_element_type=jnp.float32) + bg_ref[...]
    tg_ref[...] = jnp.where(is_var, tg_col, tg_row)
    th_col = jnp.dot(xb, whc_ref[...], preferred_element_type=jnp.float32)
    th_row = jnp.dot(xb, whr_ref[...],
                     preferred_element_type=jnp.float32) + bh_ref[...]
    th_ref[...] = jnp.where(is_var, th_col, th_row)


def _tc_proj(x13, wgc, wgr, bg, whc, whr, bh):
    blk = 4096
    grid = _cdiv(N, blk)
    full = lambda shape: pl.BlockSpec(shape, lambda i: (0, 0))
    return pl.pallas_call(
        _proj_body,
        grid=(grid,),
        in_specs=[pl.BlockSpec((blk, VF), lambda i: (i, 0)),
                  full((VF, 32)), full((CF, 32)), full((1, 32)),
                  full((VF, 32)), full((VF, 32)), full((1, 32))],
        out_specs=[pl.BlockSpec((blk, 32), lambda i: (i, 0)),
                   pl.BlockSpec((blk, 32), lambda i: (i, 0))],
        out_shape=[jax.ShapeDtypeStruct((N, 32), jnp.float32),
                   jax.ShapeDtypeStruct((N, 32), jnp.float32)],
    )(x13, wgc, wgr, bg, whc, whr, bh)


def _finalize_aggr(a0, a1, cnt_t, w2t, b2):
    cnt = jnp.sum(cnt_t, axis=0)[:, None]
    m = (a0 + a1) / jnp.maximum(cnt, 1.0)
    return (jnp.dot(m, w2t, preferred_element_type=jnp.float32)
            + b2 * (cnt > 0.0))


def _cvars_body(ag0_ref, ag1_ref, cg_ref, ah0_ref, ah1_ref, ch_ref, x_ref,
                gw2_ref, gb2_ref, hw2_ref, hb2_ref, w1a_ref, w1b_ref, w1c_ref,
                b1_ref, w2_ref, b2_ref, gmid_ref, tc_ref):
    ag = _finalize_aggr(ag0_ref[...], ag1_ref[...], cg_ref[...], gw2_ref[...],
                        gb2_ref[...])
    ah = _finalize_aggr(ah0_ref[...], ah1_ref[...], ch_ref[...], hw2_ref[...],
                        hb2_ref[...])
    h = jnp.maximum(
        jnp.dot(x_ref[...], w1a_ref[...], preferred_element_type=jnp.float32)
        + jnp.dot(ag, w1b_ref[...], preferred_element_type=jnp.float32)
        + jnp.dot(ah, w1c_ref[...], preferred_element_type=jnp.float32)
        + b1_ref[...], 0.0)
    fv = jnp.dot(h, w2_ref[...], preferred_element_type=jnp.float32) + b2_ref[...]
    tc_ref[...] = jnp.dot(fv, gmid_ref[...], preferred_element_type=jnp.float32)


def _tc_cvars(ag0, ag1, cg, ah0, ah1, ch, x13v, gw2t, gb2, hw2t, hb2, w1a,
              w1b, w1c, b1, w2t, b2, gmidt):
    blk = 4096
    grid = _cdiv(NVARS, blk)
    full = lambda r, c: pl.BlockSpec((r, c), lambda i: (0, 0))
    row32 = pl.BlockSpec((blk, 32), lambda i: (i, 0))
    cnt_spec = pl.BlockSpec((NW, blk), lambda i: (0, i))
    return pl.pallas_call(
        _cvars_body,
        grid=(grid,),
        in_specs=[row32, row32, cnt_spec, row32, row32, cnt_spec,
                  pl.BlockSpec((blk, VF), lambda i: (i, 0)),
                  full(32, 32), full(1, 32), full(32, 32), full(1, 32),
                  full(VF, 32), full(32, 32), full(32, 32), full(1, 32),
                  full(32, 32), full(1, 32), full(32, 32)],
        out_specs=pl.BlockSpec((blk, 32), lambda i: (i, 0)),
        out_shape=jax.ShapeDtypeStruct((NVARS, 32), jnp.float32),
    )(ag0, ag1, cg, ah0, ah1, ch, x13v, gw2t, gb2, hw2t, hb2, w1a, w1b, w1c,
      b1, w2t, b2, gmidt)


def _ccuts_body(x_ref, w_ref, b_ref, out_ref):
    out_ref[...] = (jnp.dot(x_ref[...], w_ref[...],
                            preferred_element_type=jnp.float32) + b_ref[...])


def _tc_ccuts(x13c, wt, b):
    return pl.pallas_call(
        _ccuts_body,
        out_shape=jax.ShapeDtypeStruct((NCUTS, 32), jnp.float32),
    )(x13c, wt, b)


def _final_body(a0_ref, a1_ref, cc_ref, x_ref, gw2_ref, gb2_ref, w1a_ref,
                w1b_ref, b1_ref, w2_ref, b2_ref, out_ref):
    ac = _finalize_aggr(a0_ref[...], a1_ref[...], cc_ref[...], gw2_ref[...],
                        gb2_ref[...])
    h = jnp.maximum(
        jnp.dot(x_ref[...], w1a_ref[...], preferred_element_type=jnp.float32)
        + jnp.dot(ac, w1b_ref[...], preferred_element_type=jnp.float32)
        + b1_ref[...], 0.0)
    out_ref[...] = (jnp.dot(h, w2_ref[...], preferred_element_type=jnp.float32)
                    + b2_ref[...])


def _tc_final(a0, a1, cc, x13c, gw2t, gb2, w1a, w1b, b1, w2t, b2):
    return pl.pallas_call(
        _final_body,
        out_shape=jax.ShapeDtypeStruct((NDST_C, 32), jnp.float32),
    )(a0, a1, cc, x13c, gw2t, gb2, w1a, w1b, b1, w2t, b2)


def kernel(x_s, edge_index_s, edge_attr_s, vars_nodes, cuts_nodes, cons_edges,
           cuts_edges, gv_W1, gv_b1, gv_W2, gv_b2, hv_W1, hv_b1, hv_W2, hv_b2,
           fv_W1, fv_b1, fv_W2, fv_b2, gc_W1, gc_b1, gc_W2, gc_b2, fc_W1,
           fc_b1, fc_W2, fc_b2):
    f32 = jnp.float32
    i32 = jnp.int32
    rows = edge_index_s[0]
    cols = edge_index_s[1]
    attr = edge_attr_s[:, 0]
    x13 = x_s[:, :VF]
    x13c = x_s[NVARS + NCONS:, :VF]

    tg, th = _tc_proj(
        x13,
        gv_W1[:, :VF].T, gv_W1[:, VF:VF + CF].T, gv_b1[None, :],
        hv_W1[:, :VF].T, hv_W1[:, VF:VF + KF].T, hv_b1[None, :])
    tcc = _tc_ccuts(x13c, gc_W1[:, :KF].T, gc_b1[None, :])

    def pad(a, n, val):
        return jnp.concatenate([a, jnp.full((n - a.shape[0],), val, a.dtype)])

    def pack3(a, b, c_, pe):
        return jnp.stack([a.reshape(pe // CHUNK, CHUNK),
                          b.reshape(pe // CHUNK, CHUNK),
                          c_.reshape(pe // CHUNK, CHUNK)], 1)

    def rep16(a, pe):
        return jnp.broadcast_to(pad(a, pe, 0.0)[:, None], (pe, 16)).reshape(
            pe // CHUNK, CHUNK, 16)

    zeros32v = jnp.zeros((NDST_V, 32), f32)
    zerocv = jnp.zeros((NDST_V,), f32)

    ie_g = pack3(pad(cols[:EC], PE_C, 0), pad(rows[:EC], PE_C, 0),
                 pad(cols[:EC], PE_C, NDST_V - 1), PE_C)
    il_g = rep16(attr[:EC], PE_C)
    acc_g, cnt_g = _edge_aggr_g(tg, ie_g, il_g, zeros32v, zerocv,
                                gv_W1[:, VF + CF])

    ie_h = pack3(pad(cols[EC:], PE_K, 0), pad(rows[EC:], PE_K, 0),
                 pad(cols[EC:], PE_K, NDST_V - 1), PE_K)
    il_h = rep16(attr[EC:], PE_K)
    acc_h, cnt_h = _edge_aggr_h(th, ie_h, il_h, zeros32v, zerocv,
                                hv_W1[:, VF + KF])

    tcv = _tc_cvars(
        acc_g[:NDST_V][:NVARS], acc_g[NDST_V:][:NVARS], cnt_g[:, :NVARS],
        acc_h[:NDST_V][:NVARS], acc_h[NDST_V:][:NVARS], cnt_h[:, :NVARS],
        x13[:NVARS],
        gv_W2.T, gv_b2[None, :], hv_W2.T, hv_b2[None, :],
        fv_W1[:, :VF].T, fv_W1[:, VF:VF + OC].T, fv_W1[:, VF + OC:].T,
        fv_b1[None, :], fv_W2.T, fv_b2[None, :],
        gc_W1[:, KF:KF + OC].T)
    tc_tab = jnp.concatenate([tcv, jnp.zeros((NCONS, 32), f32), tcc])

    ie_d = pack3(pad(rows[EC:], PE_K, 0), pad(cols[EC:], PE_K, 0),
                 pad(rows[EC:] - (NVARS + NCONS), PE_K, NDST_C - 1), PE_K)
    il_d = il_h
    acc_c, cnt_c = _edge_aggr_d(tc_tab, ie_d, il_d,
                                jnp.zeros((NDST_C, 32), f32),
                                jnp.zeros((NDST_C,), f32), gc_W1[:, KF + OC])

    x13c_pad = jnp.concatenate([x13c, jnp.zeros((NDST_C - NCUTS, VF), f32)])
    out = _tc_final(
        acc_c[:NDST_C], acc_c[NDST_C:], cnt_c,
        x13c_pad, gc_W2.T, gc_b2[None, :],
        fc_W1[:, :VF].T, fc_W1[:, VF:].T, fc_b1[None, :],
        fc_W2.T, fc_b2[None, :])
    return out[:NCUTS]


# speedup vs baseline: 1.4383x; 1.4383x over previous
import functools

import jax
import jax.numpy as jnp
from jax import lax
from jax.experimental import pallas as pl
from jax.experimental.pallas import tpu as pltpu
from jax.experimental.pallas import tpu_sc as plsc

NVARS = 25000
NCONS = 20000
NCUTS = 5000
N = NVARS + NCONS + NCUTS
EC = 1280000
EK = 320000
E = EC + EK
VF = 13
CF = 5
KF = 13
OC = 32

NCORES = 2
NSUB = 16
NW = NCORES * NSUB
CHUNK = 128

NDST_V = 25088
NDST_C = 5008
TCH_C = 316
TCH_K = 80
PE_C = NW * TCH_C * CHUNK
PE_K = NW * TCH_K * CHUNK

_GDN = lax.GatherDimensionNumbers(offset_dims=(), collapsed_slice_dims=(0,),
                                  start_index_map=(0,))


def _lane_bcast(vec16, lane):
    idx = jnp.full((16, 1), lane, jnp.int32)
    return lax.gather(vec16, idx, _GDN, (1,),
                      mode=lax.GatherScatterMode.PROMISE_IN_BOUNDS)


def _cdiv(a, b):
    return (a + b - 1) // b


def _make_edge_aggr(tchunks, ndst_pad):
    rpt = ndst_pad // NSUB
    mesh = plsc.VectorSubcoreMesh(core_axis_name="c", subcore_axis_name="s")

    @functools.partial(
        pl.kernel,
        mesh=mesh,
        compiler_params=pltpu.CompilerParams(use_tc_tiling_on_sc=False,
                                             needs_layout_passes=False),
        out_type=[
            jax.ShapeDtypeStruct((NCORES * ndst_pad, 32), jnp.float32),
            jax.ShapeDtypeStruct((NW, ndst_pad), jnp.float32),
        ],
        scratch_types=(
            [pltpu.VMEM((2, 128), jnp.int32)] * 4
            + [pltpu.VMEM((2, 128), jnp.float32)] * 2
            + [pltpu.VMEM((128,), jnp.int32)] * 2
            + [pltpu.VMEM((CHUNK, 32), jnp.float32)] * 4
            + [pltpu.VMEM((CHUNK, 32), jnp.float32)] * 4
            + [pltpu.VMEM((CHUNK, 32), jnp.float32)] * 2
            + [pltpu.VMEM((32,), jnp.float32),
               pltpu.VMEM((ndst_pad,), jnp.float32),
               pltpu.VMEM_SHARED((ndst_pad, 32), jnp.float32)]
            + [pltpu.SemaphoreType.DMA] * 12
        ),
    )
    def k(t_hbm, ie_hbm, il_hbm, z32_hbm, zc_hbm, w_hbm, out_acc, out_cnt,
          ie0, ie1, ie2, ie3, il0, il1, ds0, ds1, ra0, ra1, ra2, ra3,
          rb0, rb1, rb2, rb3, ov0, ov1, w_v, cnt_v, acc_s,
          se0, se1, se2, se3, sl0, sl1, sg0, sg1, sg2, sg3, ss0, ss1):
        c = lax.axis_index("c")
        s = lax.axis_index("s")
        wid = c * NSUB + s
        ie = (ie0, ie1, ie2, ie3)
        il = (il0, il1)
        ds = (ds0, ds1)
        ra = (ra0, ra1, ra2, ra3)
        rb = (rb0, rb1, rb2, rb3)
        ov = (ov0, ov1)
        sem_e = (se0, se1, se2, se3)
        sem_l = (sl0, sl1)
        sem_g = (sg0, sg1, sg2, sg3)
        sem_s = (ss0, ss1)

        r0 = s * rpt
        pltpu.sync_copy(z32_hbm.at[pl.ds(r0, rpt)], acc_s.at[pl.ds(r0, rpt)])
        pltpu.sync_copy(zc_hbm, cnt_v)
        pltpu.sync_copy(w_hbm, w_v)
        w_lo = w_v[pl.ds(0, 16)]
        w_hi = w_v[pl.ds(16, 16)]
        plsc.subcore_barrier()

        base = wid * tchunks

        def start_e(i, slot):
            return pltpu.async_copy(ie_hbm.at[base + i], ie[slot], sem_e[slot])

        def start_l(i, slot):
            return pltpu.async_copy(il_hbm.at[base + i], il[slot], sem_l[slot])

        def start_gathers(i, slot):
            del i
            pltpu.async_copy(t_hbm.at[ie[slot].at[0]], ra[slot], sem_g[slot])
            pltpu.async_copy(t_hbm.at[ie[slot].at[1]], rb[slot], sem_g[slot])

        def wait_gathers(slot):
            pltpu.make_async_copy(t_hbm.at[ie[slot].at[0]], ra[slot],
                                  sem_g[slot]).wait()
            pltpu.make_async_copy(t_hbm.at[ie[slot].at[1]], rb[slot],
                                  sem_g[slot]).wait()

        def wait_e(slot):
            pltpu.make_async_copy(ie_hbm.at[0], ie[slot], sem_e[slot]).wait()

        start_e(0, 0)
        start_e(1, 1)
        start_e(2, 2)
        start_l(0, 0)
        start_l(1, 1)
        wait_e(0)
        start_gathers(0, 0)
        wait_e(1)
        start_gathers(1, 1)

        def iteration(i, s4, s2):
            n4 = (s4 + 2) % 4
            @pl.when(i + 2 < tchunks)
            def _():
                wait_e(n4)
                start_gathers(i + 2, n4)

            wait_gathers(s4)
            pltpu.make_async_copy(il_hbm.at[0], il[s2], sem_l[s2]).wait()

            @pl.when(i >= 2)
            def _():
                pltpu.make_async_copy(ov[s2], acc_s.at[ds[s2]],
                                      sem_s[s2]).wait()

            rav = ra[s4]
            rbv = rb[s4]
            ovv = ov[s2]
            ilv = il[s2]

            one16 = jnp.ones((16,), jnp.float32)
            for t in range(CHUNK // 16):
                ds16 = ilv[0, pl.ds(t * 16, 16)].astype(jnp.int32)
                ds[s2][pl.ds(t * 16, 16)] = ds16
                plsc.addupdate_scatter(cnt_v, [ds16], one16)

            def group(g, carry):
                a16 = ilv[1, pl.ds(g * 16, 16)]
                gbase = g * 16
                for u in range(16):
                    e = gbase + u
                    av = _lane_bcast(a16, u)
                    v0 = rav[e, pl.ds(0, 16)] + rbv[e, pl.ds(0, 16)] + av * w_lo
                    ovv[e, pl.ds(0, 16)] = jnp.maximum(v0, 0.0)
                    v1 = (rav[e, pl.ds(16, 16)] + rbv[e, pl.ds(16, 16)]
                          + av * w_hi)
                    ovv[e, pl.ds(16, 16)] = jnp.maximum(v1, 0.0)
                return carry

            lax.fori_loop(0, CHUNK // 16, group, 0, unroll=False)

            pltpu.async_copy(ovv, acc_s.at[ds[s2]], sem_s[s2], add=True)

            @pl.when(i + 3 < tchunks)
            def _():
                start_e(i + 3, (s4 + 3) % 4)

            @pl.when(i + 2 < tchunks)
            def _():
                start_l(i + 2, s2)

        def quad_body(p, carry):
            for j in range(4):
                iteration(4 * p + j, j, j % 2)
            return carry

        lax.fori_loop(0, tchunks // 4, quad_body, 0, unroll=False)
        pltpu.make_async_copy(ov[0], acc_s.at[ds[0]], sem_s[0]).wait()
        pltpu.make_async_copy(ov[1], acc_s.at[ds[1]], sem_s[1]).wait()
        plsc.subcore_barrier()

        o0 = c * ndst_pad + r0
        pltpu.sync_copy(acc_s.at[pl.ds(r0, rpt)], out_acc.at[pl.ds(o0, rpt)])
        pltpu.sync_copy(cnt_v, out_cnt.at[wid])

    return k


_edge_aggr_g = _make_edge_aggr(TCH_C, NDST_V)
_edge_aggr_h = _make_edge_aggr(TCH_K, NDST_V)
_edge_aggr_d = _make_edge_aggr(TCH_K, NDST_C)


def _proj_body(x_ref, wgc_ref, wgr_ref, bg_ref, whc_ref, whr_ref, bh_ref,
               tg_ref, th_ref):
    xb = x_ref[...]
    row0 = pl.program_id(0) * xb.shape[0]
    gr = row0 + lax.broadcasted_iota(jnp.int32, (xb.shape[0], 1), 0)
    is_var = gr < NVARS
    tg_col = jnp.dot(xb, wgc_ref[...], preferred_element_type=jnp.float32)
    tg_row = jnp.dot(xb[:, :CF], wgr_ref[...],
                     preferred_element_type=jnp.float32) + bg_ref[...]
    tg_ref[...] = jnp.where(is_var, tg_col, tg_row)
    th_col = jnp.dot(xb, whc_ref[...], preferred_element_type=jnp.float32)
    th_row = jnp.dot(xb, whr_ref[...],
                     preferred_element_type=jnp.float32) + bh_ref[...]
    th_ref[...] = jnp.where(is_var, th_col, th_row)


def _tc_proj(x13, wgc, wgr, bg, whc, whr, bh):
    blk = 4096
    grid = _cdiv(N, blk)
    full = lambda shape: pl.BlockSpec(shape, lambda i: (0, 0))
    return pl.pallas_call(
        _proj_body,
        grid=(grid,),
        in_specs=[pl.BlockSpec((blk, VF), lambda i: (i, 0)),
                  full((VF, 32)), full((CF, 32)), full((1, 32)),
                  full((VF, 32)), full((VF, 32)), full((1, 32))],
        out_specs=[pl.BlockSpec((blk, 32), lambda i: (i, 0)),
                   pl.BlockSpec((blk, 32), lambda i: (i, 0))],
        out_shape=[jax.ShapeDtypeStruct((N, 32), jnp.float32),
                   jax.ShapeDtypeStruct((N, 32), jnp.float32)],
    )(x13, wgc, wgr, bg, whc, whr, bh)


def _finalize_aggr(a0, a1, cnt_t, w2t, b2):
    cnt = jnp.sum(cnt_t, axis=0)[:, None]
    m = (a0 + a1) / jnp.maximum(cnt, 1.0)
    return (jnp.dot(m, w2t, preferred_element_type=jnp.float32)
            + b2 * (cnt > 0.0))


def _cvars_body(ag0_ref, ag1_ref, cg_ref, ah0_ref, ah1_ref, ch_ref, x_ref,
                gw2_ref, gb2_ref, hw2_ref, hb2_ref, w1a_ref, w1b_ref, w1c_ref,
                b1_ref, w2_ref, b2_ref, gmid_ref, tc_ref):
    ag = _finalize_aggr(ag0_ref[...], ag1_ref[...], cg_ref[...], gw2_ref[...],
                        gb2_ref[...])
    ah = _finalize_aggr(ah0_ref[...], ah1_ref[...], ch_ref[...], hw2_ref[...],
                        hb2_ref[...])
    h = jnp.maximum(
        jnp.dot(x_ref[...], w1a_ref[...], preferred_element_type=jnp.float32)
        + jnp.dot(ag, w1b_ref[...], preferred_element_type=jnp.float32)
        + jnp.dot(ah, w1c_ref[...], preferred_element_type=jnp.float32)
        + b1_ref[...], 0.0)
    fv = jnp.dot(h, w2_ref[...], preferred_element_type=jnp.float32) + b2_ref[...]
    tc_ref[...] = jnp.dot(fv, gmid_ref[...], preferred_element_type=jnp.float32)


def _tc_cvars(ag0, ag1, cg, ah0, ah1, ch, x13v, gw2t, gb2, hw2t, hb2, w1a,
              w1b, w1c, b1, w2t, b2, gmidt):
    blk = 4096
    grid = _cdiv(NVARS, blk)
    full = lambda r, c: pl.BlockSpec((r, c), lambda i: (0, 0))
    row32 = pl.BlockSpec((blk, 32), lambda i: (i, 0))
    cnt_spec = pl.BlockSpec((NW, blk), lambda i: (0, i))
    return pl.pallas_call(
        _cvars_body,
        grid=(grid,),
        in_specs=[row32, row32, cnt_spec, row32, row32, cnt_spec,
                  pl.BlockSpec((blk, VF), lambda i: (i, 0)),
                  full(32, 32), full(1, 32), full(32, 32), full(1, 32),
                  full(VF, 32), full(32, 32), full(32, 32), full(1, 32),
                  full(32, 32), full(1, 32), full(32, 32)],
        out_specs=pl.BlockSpec((blk, 32), lambda i: (i, 0)),
        out_shape=jax.ShapeDtypeStruct((NVARS, 32), jnp.float32),
    )(ag0, ag1, cg, ah0, ah1, ch, x13v, gw2t, gb2, hw2t, hb2, w1a, w1b, w1c,
      b1, w2t, b2, gmidt)


def _ccuts_body(x_ref, w_ref, b_ref, out_ref):
    out_ref[...] = (jnp.dot(x_ref[...], w_ref[...],
                            preferred_element_type=jnp.float32) + b_ref[...])


def _tc_ccuts(x13c, wt, b):
    return pl.pallas_call(
        _ccuts_body,
        out_shape=jax.ShapeDtypeStruct((NCUTS, 32), jnp.float32),
    )(x13c, wt, b)


def _final_body(a0_ref, a1_ref, cc_ref, x_ref, gw2_ref, gb2_ref, w1a_ref,
                w1b_ref, b1_ref, w2_ref, b2_ref, out_ref):
    ac = _finalize_aggr(a0_ref[...], a1_ref[...], cc_ref[...], gw2_ref[...],
                        gb2_ref[...])
    h = jnp.maximum(
        jnp.dot(x_ref[...], w1a_ref[...], preferred_element_type=jnp.float32)
        + jnp.dot(ac, w1b_ref[...], preferred_element_type=jnp.float32)
        + b1_ref[...], 0.0)
    out_ref[...] = (jnp.dot(h, w2_ref[...], preferred_element_type=jnp.float32)
                    + b2_ref[...])


def _tc_final(a0, a1, cc, x13c, gw2t, gb2, w1a, w1b, b1, w2t, b2):
    return pl.pallas_call(
        _final_body,
        out_shape=jax.ShapeDtypeStruct((NDST_C, 32), jnp.float32),
    )(a0, a1, cc, x13c, gw2t, gb2, w1a, w1b, b1, w2t, b2)


def kernel(x_s, edge_index_s, edge_attr_s, vars_nodes, cuts_nodes, cons_edges,
           cuts_edges, gv_W1, gv_b1, gv_W2, gv_b2, hv_W1, hv_b1, hv_W2, hv_b2,
           fv_W1, fv_b1, fv_W2, fv_b2, gc_W1, gc_b1, gc_W2, gc_b2, fc_W1,
           fc_b1, fc_W2, fc_b2):
    f32 = jnp.float32
    i32 = jnp.int32
    rows = edge_index_s[0]
    cols = edge_index_s[1]
    attr = edge_attr_s[:, 0]
    x13 = x_s[:, :VF]
    x13c = x_s[NVARS + NCONS:, :VF]

    tg, th = _tc_proj(
        x13,
        gv_W1[:, :VF].T, gv_W1[:, VF:VF + CF].T, gv_b1[None, :],
        hv_W1[:, :VF].T, hv_W1[:, VF:VF + KF].T, hv_b1[None, :])
    tcc = _tc_ccuts(x13c, gc_W1[:, :KF].T, gc_b1[None, :])

    def pad(a, n, val):
        return jnp.concatenate([a, jnp.full((n - a.shape[0],), val, a.dtype)])

    def pack2(a, b, pe):
        return jnp.stack([a, b], 1).reshape(pe // CHUNK, CHUNK, 2
                                            ).transpose(0, 2, 1)

    zeros32v = jnp.zeros((NDST_V, 32), f32)
    zerocv = jnp.zeros((NDST_V,), f32)

    ie_g = pack2(pad(cols[:EC], PE_C, 0), pad(rows[:EC], PE_C, 0), PE_C)
    il_g = pack2(pad(cols[:EC].astype(f32), PE_C, float(NDST_V - 1)),
                 pad(attr[:EC], PE_C, 0.0), PE_C)
    acc_g, cnt_g = _edge_aggr_g(tg, ie_g, il_g, zeros32v, zerocv,
                                gv_W1[:, VF + CF])

    ie_h = pack2(pad(cols[EC:], PE_K, 0), pad(rows[EC:], PE_K, 0), PE_K)
    il_h = pack2(pad(cols[EC:].astype(f32), PE_K, float(NDST_V - 1)),
                 pad(attr[EC:], PE_K, 0.0), PE_K)
    acc_h, cnt_h = _edge_aggr_h(th, ie_h, il_h, zeros32v, zerocv,
                                hv_W1[:, VF + KF])

    tcv = _tc_cvars(
        acc_g[:NDST_V][:NVARS], acc_g[NDST_V:][:NVARS], cnt_g[:, :NVARS],
        acc_h[:NDST_V][:NVARS], acc_h[NDST_V:][:NVARS], cnt_h[:, :NVARS],
        x13[:NVARS],
        gv_W2.T, gv_b2[None, :], hv_W2.T, hv_b2[None, :],
        fv_W1[:, :VF].T, fv_W1[:, VF:VF + OC].T, fv_W1[:, VF + OC:].T,
        fv_b1[None, :], fv_W2.T, fv_b2[None, :],
        gc_W1[:, KF:KF + OC].T)
    tc_tab = jnp.concatenate([tcv, jnp.zeros((NCONS, 32), f32), tcc])

    ie_d = pack2(pad(rows[EC:], PE_K, 0), pad(cols[EC:], PE_K, 0), PE_K)
    il_d = pack2(pad((rows[EC:] - (NVARS + NCONS)).astype(f32), PE_K,
                     float(NDST_C - 1)),
                 pad(attr[EC:], PE_K, 0.0), PE_K)
    acc_c, cnt_c = _edge_aggr_d(tc_tab, ie_d, il_d,
                                jnp.zeros((NDST_C, 32), f32),
                                jnp.zeros((NDST_C,), f32), gc_W1[:, KF + OC])

    x13c_pad = jnp.concatenate([x13c, jnp.zeros((NDST_C - NCUTS, VF), f32)])
    out = _tc_final(
        acc_c[:NDST_C], acc_c[NDST_C:], cnt_c,
        x13c_pad, gc_W2.T, gc_b2[None, :],
        fc_W1[:, :VF].T, fc_W1[:, VF:].T, fc_b1[None, :],
        fc_W2.T, fc_b2[None, :])
    return out[:NCUTS]


# speedup vs baseline: 1.6438x; 1.1429x over previous
import functools

import jax
import jax.numpy as jnp
from jax import lax
from jax.experimental import pallas as pl
from jax.experimental.pallas import tpu as pltpu
from jax.experimental.pallas import tpu_sc as plsc

NVARS = 25000
NCONS = 20000
NCUTS = 5000
N = NVARS + NCONS + NCUTS
EC = 1280000
EK = 320000
E = EC + EK
VF = 13
CF = 5
KF = 13
OC = 32

NCORES = 2
NSUB = 16
NW = NCORES * NSUB
CHUNK = 128

NDST_V = 25088
NDST_C = 5008
TCH_C = 314
TCH_K = 80
PE_C = NW * TCH_C * CHUNK
PE_K = NW * TCH_K * CHUNK

_GDN = lax.GatherDimensionNumbers(offset_dims=(), collapsed_slice_dims=(0,),
                                  start_index_map=(0,))


def _lane_bcast(vec16, lane):
    idx = jnp.full((16, 1), lane, jnp.int32)
    return lax.gather(vec16, idx, _GDN, (1,),
                      mode=lax.GatherScatterMode.PROMISE_IN_BOUNDS)


def _cdiv(a, b):
    return (a + b - 1) // b


def _make_edge_aggr(tchunks, ndst_pad):
    rpt = ndst_pad // NSUB
    mesh = plsc.VectorSubcoreMesh(core_axis_name="c", subcore_axis_name="s")

    @functools.partial(
        pl.kernel,
        mesh=mesh,
        compiler_params=pltpu.CompilerParams(use_tc_tiling_on_sc=False,
                                             needs_layout_passes=False),
        out_type=[
            jax.ShapeDtypeStruct((NCORES * ndst_pad, 32), jnp.float32),
            jax.ShapeDtypeStruct((NW, ndst_pad), jnp.float32),
        ],
        scratch_types=[
            pltpu.VMEM((2, 128), jnp.int32),
            pltpu.VMEM((2, 128), jnp.int32),
            pltpu.VMEM((2, 128), jnp.float32),
            pltpu.VMEM((2, 128), jnp.float32),
            pltpu.VMEM((128,), jnp.int32),
            pltpu.VMEM((128,), jnp.int32),
            pltpu.VMEM((CHUNK, 32), jnp.float32),
            pltpu.VMEM((CHUNK, 32), jnp.float32),
            pltpu.VMEM((CHUNK, 32), jnp.float32),
            pltpu.VMEM((CHUNK, 32), jnp.float32),
            pltpu.VMEM((CHUNK, 32), jnp.float32),
            pltpu.VMEM((CHUNK, 32), jnp.float32),
            pltpu.VMEM((32,), jnp.float32),
            pltpu.VMEM((ndst_pad,), jnp.float32),
            pltpu.VMEM_SHARED((ndst_pad, 32), jnp.float32),
            pltpu.SemaphoreType.DMA,
            pltpu.SemaphoreType.DMA,
            pltpu.SemaphoreType.DMA,
            pltpu.SemaphoreType.DMA,
            pltpu.SemaphoreType.DMA,
            pltpu.SemaphoreType.DMA,
            pltpu.SemaphoreType.DMA,
            pltpu.SemaphoreType.DMA,
        ],
    )
    def k(t_hbm, ie_hbm, il_hbm, z32_hbm, zc_hbm, w_hbm, out_acc, out_cnt,
          ie0, ie1, il0, il1, ds0, ds1, ra0, ra1, rb0, rb1, ov0, ov1, w_v,
          cnt_v, acc_s, se0, se1, sl0, sl1, sg0, sg1, ss0, ss1):
        c = lax.axis_index("c")
        s = lax.axis_index("s")
        wid = c * NSUB + s
        ie = (ie0, ie1)
        il = (il0, il1)
        ds = (ds0, ds1)
        ra = (ra0, ra1)
        rb = (rb0, rb1)
        ov = (ov0, ov1)
        sem_e = (se0, se1)
        sem_l = (sl0, sl1)
        sem_g = (sg0, sg1)
        sem_s = (ss0, ss1)

        r0 = s * rpt
        pltpu.sync_copy(z32_hbm.at[pl.ds(r0, rpt)], acc_s.at[pl.ds(r0, rpt)])
        pltpu.sync_copy(zc_hbm, cnt_v)
        pltpu.sync_copy(w_hbm, w_v)
        w_lo = w_v[pl.ds(0, 16)]
        w_hi = w_v[pl.ds(16, 16)]
        plsc.subcore_barrier()

        base = wid * tchunks

        def start_e(i, slot):
            return pltpu.async_copy(ie_hbm.at[base + i], ie[slot], sem_e[slot])

        def start_l(i, slot):
            return pltpu.async_copy(il_hbm.at[base + i], il[slot], sem_l[slot])

        def start_gathers(i, slot):
            del i
            pltpu.async_copy(t_hbm.at[ie[slot].at[0]], ra[slot], sem_g[slot])
            pltpu.async_copy(t_hbm.at[ie[slot].at[1]], rb[slot], sem_g[slot])

        def wait_gathers(slot):
            pltpu.make_async_copy(t_hbm.at[ie[slot].at[0]], ra[slot],
                                  sem_g[slot]).wait()
            pltpu.make_async_copy(t_hbm.at[ie[slot].at[1]], rb[slot],
                                  sem_g[slot]).wait()

        start_e(0, 0)
        start_l(0, 0)
        start_e(1, 1)
        start_l(1, 1)
        pltpu.make_async_copy(ie_hbm.at[0], ie[0], sem_e[0]).wait()
        start_gathers(0, 0)

        def iteration(i, slot):
            oslot = 1 - slot
            @pl.when(i + 1 < tchunks)
            def _():
                pltpu.make_async_copy(ie_hbm.at[0], ie[oslot],
                                      sem_e[oslot]).wait()
                start_gathers(i + 1, oslot)

            wait_gathers(slot)
            pltpu.make_async_copy(il_hbm.at[0], il[slot], sem_l[slot]).wait()

            @pl.when(i >= 2)
            def _():
                pltpu.make_async_copy(ov[slot], acc_s.at[ds[slot]],
                                      sem_s[slot]).wait()

            rav = ra[slot]
            rbv = rb[slot]
            ovv = ov[slot]
            ilv = il[slot]

            one16 = jnp.ones((16,), jnp.float32)
            for t in range(CHUNK // 16):
                ds16 = ilv[0, pl.ds(t * 16, 16)].astype(jnp.int32)
                ds[slot][pl.ds(t * 16, 16)] = ds16
                plsc.addupdate_scatter(cnt_v, [ds16], one16)

            def group(g, carry):
                a16 = ilv[1, pl.ds(g * 16, 16)]
                gbase = g * 16
                for u in range(16):
                    e = gbase + u
                    av = _lane_bcast(a16, u)
                    v0 = rav[e, pl.ds(0, 16)] + rbv[e, pl.ds(0, 16)] + av * w_lo
                    ovv[e, pl.ds(0, 16)] = jnp.maximum(v0, 0.0)
                    v1 = (rav[e, pl.ds(16, 16)] + rbv[e, pl.ds(16, 16)]
                          + av * w_hi)
                    ovv[e, pl.ds(16, 16)] = jnp.maximum(v1, 0.0)
                return carry

            lax.fori_loop(0, CHUNK // 16, group, 0, unroll=False)

            pltpu.async_copy(ovv, acc_s.at[ds[slot]], sem_s[slot], add=True)

            @pl.when(i + 2 < tchunks)
            def _():
                start_e(i + 2, slot)
                start_l(i + 2, slot)

        def pair_body(p, carry):
            iteration(2 * p, 0)
            iteration(2 * p + 1, 1)
            return carry

        lax.fori_loop(0, tchunks // 2, pair_body, 0, unroll=False)
        pltpu.make_async_copy(ov[0], acc_s.at[ds[0]], sem_s[0]).wait()
        pltpu.make_async_copy(ov[1], acc_s.at[ds[1]], sem_s[1]).wait()
        plsc.subcore_barrier()

        o0 = c * ndst_pad + r0
        pltpu.sync_copy(acc_s.at[pl.ds(r0, rpt)], out_acc.at[pl.ds(o0, rpt)])
        pltpu.sync_copy(cnt_v, out_cnt.at[wid])

    return k


_edge_aggr_g = _make_edge_aggr(TCH_C, NDST_V)
_edge_aggr_h = _make_edge_aggr(TCH_K, NDST_V)
_edge_aggr_d = _make_edge_aggr(TCH_K, NDST_C)


def _proj_body(x_ref, wgc_ref, wgr_ref, bg_ref, whc_ref, whr_ref, bh_ref,
               tg_ref, th_ref):
    xb = x_ref[...]
    row0 = pl.program_id(0) * xb.shape[0]
    gr = row0 + lax.broadcasted_iota(jnp.int32, (xb.shape[0], 1), 0)
    is_var = gr < NVARS
    tg_col = jnp.dot(xb, wgc_ref[...], preferred_element_type=jnp.float32)
    tg_row = jnp.dot(xb[:, :CF], wgr_ref[...],
                     preferred_element_type=jnp.float32) + bg_ref[...]
    tg_ref[...] = jnp.where(is_var, tg_col, tg_row)
    th_col = jnp.dot(xb, whc_ref[...], preferred_element_type=jnp.float32)
    th_row = jnp.dot(xb, whr_ref[...],
                     preferred_element_type=jnp.float32) + bh_ref[...]
    th_ref[...] = jnp.where(is_var, th_col, th_row)


def _tc_proj(x13, wgc, wgr, bg, whc, whr, bh):
    blk = 4096
    grid = _cdiv(N, blk)
    full = lambda shape: pl.BlockSpec(shape, lambda i: (0, 0))
    return pl.pallas_call(
        _proj_body,
        grid=(grid,),
        in_specs=[pl.BlockSpec((blk, VF), lambda i: (i, 0)),
                  full((VF, 32)), full((CF, 32)), full((1, 32)),
                  full((VF, 32)), full((VF, 32)), full((1, 32))],
        out_specs=[pl.BlockSpec((blk, 32), lambda i: (i, 0)),
                   pl.BlockSpec((blk, 32), lambda i: (i, 0))],
        out_shape=[jax.ShapeDtypeStruct((N, 32), jnp.float32),
                   jax.ShapeDtypeStruct((N, 32), jnp.float32)],
    )(x13, wgc, wgr, bg, whc, whr, bh)


def _finalize_aggr(a0, a1, cnt_t, w2t, b2):
    cnt = jnp.sum(cnt_t, axis=0)[:, None]
    m = (a0 + a1) / jnp.maximum(cnt, 1.0)
    return (jnp.dot(m, w2t, preferred_element_type=jnp.float32)
            + b2 * (cnt > 0.0))


def _cvars_body(ag0_ref, ag1_ref, cg_ref, ah0_ref, ah1_ref, ch_ref, x_ref,
                gw2_ref, gb2_ref, hw2_ref, hb2_ref, w1a_ref, w1b_ref, w1c_ref,
                b1_ref, w2_ref, b2_ref, gmid_ref, tc_ref):
    ag = _finalize_aggr(ag0_ref[...], ag1_ref[...], cg_ref[...], gw2_ref[...],
                        gb2_ref[...])
    ah = _finalize_aggr(ah0_ref[...], ah1_ref[...], ch_ref[...], hw2_ref[...],
                        hb2_ref[...])
    h = jnp.maximum(
        jnp.dot(x_ref[...], w1a_ref[...], preferred_element_type=jnp.float32)
        + jnp.dot(ag, w1b_ref[...], preferred_element_type=jnp.float32)
        + jnp.dot(ah, w1c_ref[...], preferred_element_type=jnp.float32)
        + b1_ref[...], 0.0)
    fv = jnp.dot(h, w2_ref[...], preferred_element_type=jnp.float32) + b2_ref[...]
    tc_ref[...] = jnp.dot(fv, gmid_ref[...], preferred_element_type=jnp.float32)


def _tc_cvars(ag0, ag1, cg, ah0, ah1, ch, x13v, gw2t, gb2, hw2t, hb2, w1a,
              w1b, w1c, b1, w2t, b2, gmidt):
    blk = 4096
    grid = _cdiv(NVARS, blk)
    full = lambda r, c: pl.BlockSpec((r, c), lambda i: (0, 0))
    row32 = pl.BlockSpec((blk, 32), lambda i: (i, 0))
    cnt_spec = pl.BlockSpec((NW, blk), lambda i: (0, i))
    return pl.pallas_call(
        _cvars_body,
        grid=(grid,),
        in_specs=[row32, row32, cnt_spec, row32, row32, cnt_spec,
                  pl.BlockSpec((blk, VF), lambda i: (i, 0)),
                  full(32, 32), full(1, 32), full(32, 32), full(1, 32),
                  full(VF, 32), full(32, 32), full(32, 32), full(1, 32),
                  full(32, 32), full(1, 32), full(32, 32)],
        out_specs=pl.BlockSpec((blk, 32), lambda i: (i, 0)),
        out_shape=jax.ShapeDtypeStruct((NVARS, 32), jnp.float32),
    )(ag0, ag1, cg, ah0, ah1, ch, x13v, gw2t, gb2, hw2t, hb2, w1a, w1b, w1c,
      b1, w2t, b2, gmidt)


def _ccuts_body(x_ref, w_ref, b_ref, out_ref):
    out_ref[...] = (jnp.dot(x_ref[...], w_ref[...],
                            preferred_element_type=jnp.float32) + b_ref[...])


def _tc_ccuts(x13c, wt, b):
    return pl.pallas_call(
        _ccuts_body,
        out_shape=jax.ShapeDtypeStruct((NCUTS, 32), jnp.float32),
    )(x13c, wt, b)


def _final_body(a0_ref, a1_ref, cc_ref, x_ref, gw2_ref, gb2_ref, w1a_ref,
                w1b_ref, b1_ref, w2_ref, b2_ref, out_ref):
    ac = _finalize_aggr(a0_ref[...], a1_ref[...], cc_ref[...], gw2_ref[...],
                        gb2_ref[...])
    h = jnp.maximum(
        jnp.dot(x_ref[...], w1a_ref[...], preferred_element_type=jnp.float32)
        + jnp.dot(ac, w1b_ref[...], preferred_element_type=jnp.float32)
        + b1_ref[...], 0.0)
    out_ref[...] = (jnp.dot(h, w2_ref[...], preferred_element_type=jnp.float32)
                    + b2_ref[...])


def _tc_final(a0, a1, cc, x13c, gw2t, gb2, w1a, w1b, b1, w2t, b2):
    return pl.pallas_call(
        _final_body,
        out_shape=jax.ShapeDtypeStruct((NDST_C, 32), jnp.float32),
    )(a0, a1, cc, x13c, gw2t, gb2, w1a, w1b, b1, w2t, b2)


def kernel(x_s, edge_index_s, edge_attr_s, vars_nodes, cuts_nodes, cons_edges,
           cuts_edges, gv_W1, gv_b1, gv_W2, gv_b2, hv_W1, hv_b1, hv_W2, hv_b2,
           fv_W1, fv_b1, fv_W2, fv_b2, gc_W1, gc_b1, gc_W2, gc_b2, fc_W1,
           fc_b1, fc_W2, fc_b2):
    f32 = jnp.float32
    i32 = jnp.int32
    rows = edge_index_s[0]
    cols = edge_index_s[1]
    attr = edge_attr_s[:, 0]
    x13 = x_s[:, :VF]
    x13c = x_s[NVARS + NCONS:, :VF]

    tg, th = _tc_proj(
        x13,
        gv_W1[:, :VF].T, gv_W1[:, VF:VF + CF].T, gv_b1[None, :],
        hv_W1[:, :VF].T, hv_W1[:, VF:VF + KF].T, hv_b1[None, :])
    tcc = _tc_ccuts(x13c, gc_W1[:, :KF].T, gc_b1[None, :])

    def pad(a, n, val):
        return jnp.concatenate([a, jnp.full((n - a.shape[0],), val, a.dtype)])

    def pack2(a, b, pe):
        return jnp.stack([a, b], 1).reshape(pe // CHUNK, CHUNK, 2
                                            ).transpose(0, 2, 1)

    zeros32v = jnp.zeros((NDST_V, 32), f32)
    zerocv = jnp.zeros((NDST_V,), f32)

    ie_g = pack2(pad(cols[:EC], PE_C, 0), pad(rows[:EC], PE_C, 0), PE_C)
    il_g = pack2(pad(cols[:EC].astype(f32), PE_C, float(NDST_V - 1)),
                 pad(attr[:EC], PE_C, 0.0), PE_C)
    acc_g, cnt_g = _edge_aggr_g(tg, ie_g, il_g, zeros32v, zerocv,
                                gv_W1[:, VF + CF])

    ie_h = pack2(pad(cols[EC:], PE_K, 0), pad(rows[EC:], PE_K, 0), PE_K)
    il_h = pack2(pad(cols[EC:].astype(f32), PE_K, float(NDST_V - 1)),
                 pad(attr[EC:], PE_K, 0.0), PE_K)
    acc_h, cnt_h = _edge_aggr_h(th, ie_h, il_h, zeros32v, zerocv,
                                hv_W1[:, VF + KF])

    tcv = _tc_cvars(
        acc_g[:NDST_V][:NVARS], acc_g[NDST_V:][:NVARS], cnt_g[:, :NVARS],
        acc_h[:NDST_V][:NVARS], acc_h[NDST_V:][:NVARS], cnt_h[:, :NVARS],
        x13[:NVARS],
        gv_W2.T, gv_b2[None, :], hv_W2.T, hv_b2[None, :],
        fv_W1[:, :VF].T, fv_W1[:, VF:VF + OC].T, fv_W1[:, VF + OC:].T,
        fv_b1[None, :], fv_W2.T, fv_b2[None, :],
        gc_W1[:, KF:KF + OC].T)
    tc_tab = jnp.concatenate([tcv, jnp.zeros((NCONS, 32), f32), tcc])

    ie_d = pack2(pad(rows[EC:], PE_K, 0), pad(cols[EC:], PE_K, 0), PE_K)
    il_d = pack2(pad((rows[EC:] - (NVARS + NCONS)).astype(f32), PE_K,
                     float(NDST_C - 1)),
                 pad(attr[EC:], PE_K, 0.0), PE_K)
    acc_c, cnt_c = _edge_aggr_d(tc_tab, ie_d, il_d,
                                jnp.zeros((NDST_C, 32), f32),
                                jnp.zeros((NDST_C,), f32), gc_W1[:, KF + OC])

    x13c_pad = jnp.concatenate([x13c, jnp.zeros((NDST_C - NCUTS, VF), f32)])
    out = _tc_final(
        acc_c[:NDST_C], acc_c[NDST_C:], cnt_c,
        x13c_pad, gc_W2.T, gc_b2[None, :],
        fc_W1[:, :VF].T, fc_W1[:, VF:].T, fc_b1[None, :],
        fc_W2.T, fc_b2[None, :])
    return out[:NCUTS]


# speedup vs baseline: 1.7097x; 1.0401x over previous
import functools

import jax
import jax.numpy as jnp
from jax import lax
from jax.experimental import pallas as pl
from jax.experimental.pallas import tpu as pltpu
from jax.experimental.pallas import tpu_sc as plsc

NVARS = 25000
NCONS = 20000
NCUTS = 5000
N = NVARS + NCONS + NCUTS
EC = 1280000
EK = 320000
E = EC + EK
VF = 13
CF = 5
KF = 13
OC = 32

NCORES = 2
NSUB = 16
NW = NCORES * NSUB
CHUNK = 128

NDST_V = 25088
NDST_C = 5008
TCH_C = 314
TCH_K = 80
PE_C = NW * TCH_C * CHUNK
PE_K = NW * TCH_K * CHUNK

_GDN = lax.GatherDimensionNumbers(offset_dims=(), collapsed_slice_dims=(0,),
                                  start_index_map=(0,))


def _lane_bcast(vec16, lane):
    idx = jnp.full((16, 1), lane, jnp.int32)
    return lax.gather(vec16, idx, _GDN, (1,),
                      mode=lax.GatherScatterMode.PROMISE_IN_BOUNDS)


def _cdiv(a, b):
    return (a + b - 1) // b


def _make_edge_aggr(tchunks, ndst_pad):
    rpt = ndst_pad // NSUB
    mesh = plsc.VectorSubcoreMesh(core_axis_name="c", subcore_axis_name="s")

    @functools.partial(
        pl.kernel,
        mesh=mesh,
        compiler_params=pltpu.CompilerParams(use_tc_tiling_on_sc=False,
                                             needs_layout_passes=False),
        out_type=[
            jax.ShapeDtypeStruct((NCORES * ndst_pad, 32), jnp.float32),
            jax.ShapeDtypeStruct((NW, ndst_pad), jnp.float32),
        ],
        scratch_types=[
            pltpu.VMEM((2, 128), jnp.int32),
            pltpu.VMEM((2, 128), jnp.int32),
            pltpu.VMEM((2, 128), jnp.float32),
            pltpu.VMEM((2, 128), jnp.float32),
            pltpu.VMEM((128,), jnp.int32),
            pltpu.VMEM((128,), jnp.int32),
            pltpu.VMEM((CHUNK, 32), jnp.float32),
            pltpu.VMEM((CHUNK, 32), jnp.float32),
            pltpu.VMEM((CHUNK, 32), jnp.float32),
            pltpu.VMEM((CHUNK, 32), jnp.float32),
            pltpu.VMEM((CHUNK, 32), jnp.float32),
            pltpu.VMEM((CHUNK, 32), jnp.float32),
            pltpu.VMEM((32,), jnp.float32),
            pltpu.VMEM((ndst_pad,), jnp.float32),
            pltpu.VMEM_SHARED((ndst_pad, 32), jnp.float32),
            pltpu.SemaphoreType.DMA,
            pltpu.SemaphoreType.DMA,
            pltpu.SemaphoreType.DMA,
            pltpu.SemaphoreType.DMA,
            pltpu.SemaphoreType.DMA,
            pltpu.SemaphoreType.DMA,
            pltpu.SemaphoreType.DMA,
            pltpu.SemaphoreType.DMA,
        ],
    )
    def k(t_hbm, ie_hbm, il_hbm, z32_hbm, zc_hbm, w_hbm, out_acc, out_cnt,
          ie0, ie1, il0, il1, ds0, ds1, ra0, ra1, rb0, rb1, ov0, ov1, w_v,
          cnt_v, acc_s, se0, se1, sl0, sl1, sg0, sg1, ss0, ss1):
        c = lax.axis_index("c")
        s = lax.axis_index("s")
        wid = c * NSUB + s
        ie = (ie0, ie1)
        il = (il0, il1)
        ds = (ds0, ds1)
        ra = (ra0, ra1)
        rb = (rb0, rb1)
        ov = (ov0, ov1)
        sem_e = (se0, se1)
        sem_l = (sl0, sl1)
        sem_g = (sg0, sg1)
        sem_s = (ss0, ss1)

        r0 = s * rpt
        pltpu.sync_copy(z32_hbm.at[pl.ds(r0, rpt)], acc_s.at[pl.ds(r0, rpt)])
        pltpu.sync_copy(zc_hbm, cnt_v)
        pltpu.sync_copy(w_hbm, w_v)
        w_lo = w_v[pl.ds(0, 16)]
        w_hi = w_v[pl.ds(16, 16)]
        plsc.subcore_barrier()

        base = wid * tchunks

        def start_e(i, slot):
            return pltpu.async_copy(ie_hbm.at[base + i], ie[slot], sem_e[slot])

        def start_l(i, slot):
            return pltpu.async_copy(il_hbm.at[base + i], il[slot], sem_l[slot])

        def start_gathers(i, slot):
            del i
            pltpu.async_copy(t_hbm.at[ie[slot].at[0]], ra[slot], sem_g[slot])
            pltpu.async_copy(t_hbm.at[ie[slot].at[1]], rb[slot], sem_g[slot])

        def wait_gathers(slot):
            pltpu.make_async_copy(t_hbm.at[ie[slot].at[0]], ra[slot],
                                  sem_g[slot]).wait()
            pltpu.make_async_copy(t_hbm.at[ie[slot].at[1]], rb[slot],
                                  sem_g[slot]).wait()

        start_e(0, 0)
        start_l(0, 0)
        start_e(1, 1)
        start_l(1, 1)
        pltpu.make_async_copy(ie_hbm.at[0], ie[0], sem_e[0]).wait()
        start_gathers(0, 0)

        def iteration(i, slot):
            oslot = 1 - slot
            @pl.when(i + 1 < tchunks)
            def _():
                pltpu.make_async_copy(ie_hbm.at[0], ie[oslot],
                                      sem_e[oslot]).wait()
                start_gathers(i + 1, oslot)

            wait_gathers(slot)
            pltpu.make_async_copy(il_hbm.at[0], il[slot], sem_l[slot]).wait()

            @pl.when(i >= 2)
            def _():
                pltpu.make_async_copy(ov[slot], acc_s.at[ds[slot]],
                                      sem_s[slot]).wait()

            rav = ra[slot]
            rbv = rb[slot]
            ovv = ov[slot]
            ilv = il[slot]

            one16 = jnp.ones((16,), jnp.float32)
            for t in range(CHUNK // 16):
                ds16 = ilv[0, pl.ds(t * 16, 16)].astype(jnp.int32)
                ds[slot][pl.ds(t * 16, 16)] = ds16
                plsc.addupdate_scatter(cnt_v, [ds16], one16)

            def group(g, carry):
                a16 = ilv[1, pl.ds(g * 16, 16)]
                gbase = g * 16
                for u in range(16):
                    e = gbase + u
                    av = _lane_bcast(a16, u)
                    v0 = rav[e, pl.ds(0, 16)] + rbv[e, pl.ds(0, 16)] + av * w_lo
                    ovv[e, pl.ds(0, 16)] = jnp.maximum(v0, 0.0)
                    v1 = (rav[e, pl.ds(16, 16)] + rbv[e, pl.ds(16, 16)]
                          + av * w_hi)
                    ovv[e, pl.ds(16, 16)] = jnp.maximum(v1, 0.0)
                return carry

            lax.fori_loop(0, CHUNK // 16, group, 0, unroll=False)

            pltpu.async_copy(ovv, acc_s.at[ds[slot]], sem_s[slot], add=True)

            @pl.when(i + 2 < tchunks)
            def _():
                start_e(i + 2, slot)
                start_l(i + 2, slot)

        def pair_body(p, carry):
            iteration(2 * p, 0)
            iteration(2 * p + 1, 1)
            return carry

        lax.fori_loop(0, tchunks // 2, pair_body, 0, unroll=False)
        pltpu.make_async_copy(ov[0], acc_s.at[ds[0]], sem_s[0]).wait()
        pltpu.make_async_copy(ov[1], acc_s.at[ds[1]], sem_s[1]).wait()
        plsc.subcore_barrier()

        o0 = c * ndst_pad + r0
        pltpu.sync_copy(acc_s.at[pl.ds(r0, rpt)], out_acc.at[pl.ds(o0, rpt)])
        pltpu.sync_copy(cnt_v, out_cnt.at[wid])

    return k


_edge_aggr_g = _make_edge_aggr(TCH_C, NDST_V)
_edge_aggr_h = _make_edge_aggr(TCH_K, NDST_V)
_edge_aggr_d = _make_edge_aggr(TCH_K, NDST_C)


def _proj_body(x_ref, wgc_ref, wgr_ref, bg_ref, whc_ref, whr_ref, bh_ref,
               tg_ref, th_ref):
    xb = x_ref[...]
    row0 = pl.program_id(0) * xb.shape[0]
    gr = row0 + lax.broadcasted_iota(jnp.int32, (xb.shape[0], 1), 0)
    is_var = gr < NVARS
    tg_col = jnp.dot(xb, wgc_ref[...], preferred_element_type=jnp.float32)
    tg_row = jnp.dot(xb[:, :CF], wgr_ref[...],
                     preferred_element_type=jnp.float32) + bg_ref[...]
    tg_ref[...] = jnp.where(is_var, tg_col, tg_row)
    th_col = jnp.dot(xb, whc_ref[...], preferred_element_type=jnp.float32)
    th_row = jnp.dot(xb, whr_ref[...],
                     preferred_element_type=jnp.float32) + bh_ref[...]
    th_ref[...] = jnp.where(is_var, th_col, th_row)


def _tc_proj(x13, wgc, wgr, bg, whc, whr, bh):
    blk = 4096
    grid = _cdiv(N, blk)
    full = lambda shape: pl.BlockSpec(shape, lambda i: (0, 0))
    return pl.pallas_call(
        _proj_body,
        grid=(grid,),
        in_specs=[pl.BlockSpec((blk, VF), lambda i: (i, 0)),
                  full((VF, 32)), full((CF, 32)), full((1, 32)),
                  full((VF, 32)), full((VF, 32)), full((1, 32))],
        out_specs=[pl.BlockSpec((blk, 32), lambda i: (i, 0)),
                   pl.BlockSpec((blk, 32), lambda i: (i, 0))],
        out_shape=[jax.ShapeDtypeStruct((N, 32), jnp.float32),
                   jax.ShapeDtypeStruct((N, 32), jnp.float32)],
    )(x13, wgc, wgr, bg, whc, whr, bh)


def _finalize_aggr(a0, a1, cnt_t, w2t, b2):
    cnt = jnp.sum(cnt_t, axis=0)[:, None]
    m = (a0 + a1) / jnp.maximum(cnt, 1.0)
    return (jnp.dot(m, w2t, preferred_element_type=jnp.float32)
            + b2 * (cnt > 0.0))


def _cvars_body(ag0_ref, ag1_ref, cg_ref, ah0_ref, ah1_ref, ch_ref, x_ref,
                gw2_ref, gb2_ref, hw2_ref, hb2_ref, w1a_ref, w1b_ref, w1c_ref,
                b1_ref, w2_ref, b2_ref, gmid_ref, tc_ref):
    ag = _finalize_aggr(ag0_ref[...], ag1_ref[...], cg_ref[...], gw2_ref[...],
                        gb2_ref[...])
    ah = _finalize_aggr(ah0_ref[...], ah1_ref[...], ch_ref[...], hw2_ref[...],
                        hb2_ref[...])
    h = jnp.maximum(
        jnp.dot(x_ref[...], w1a_ref[...], preferred_element_type=jnp.float32)
        + jnp.dot(ag, w1b_ref[...], preferred_element_type=jnp.float32)
        + jnp.dot(ah, w1c_ref[...], preferred_element_type=jnp.float32)
        + b1_ref[...], 0.0)
    fv = jnp.dot(h, w2_ref[...], preferred_element_type=jnp.float32) + b2_ref[...]
    tc_ref[...] = jnp.dot(fv, gmid_ref[...], preferred_element_type=jnp.float32)


def _tc_cvars(ag0, ag1, cg, ah0, ah1, ch, x13v, gw2t, gb2, hw2t, hb2, w1a,
              w1b, w1c, b1, w2t, b2, gmidt):
    blk = 4096
    grid = _cdiv(NVARS, blk)
    full = lambda r, c: pl.BlockSpec((r, c), lambda i: (0, 0))
    row32 = pl.BlockSpec((blk, 32), lambda i: (i, 0))
    cnt_spec = pl.BlockSpec((NW, blk), lambda i: (0, i))
    return pl.pallas_call(
        _cvars_body,
        grid=(grid,),
        compiler_params=pltpu.CompilerParams(allow_input_fusion=[True] * 18),
        in_specs=[row32, row32, cnt_spec, row32, row32, cnt_spec,
                  pl.BlockSpec((blk, VF), lambda i: (i, 0)),
                  full(32, 32), full(1, 32), full(32, 32), full(1, 32),
                  full(VF, 32), full(32, 32), full(32, 32), full(1, 32),
                  full(32, 32), full(1, 32), full(32, 32)],
        out_specs=pl.BlockSpec((blk, 32), lambda i: (i, 0)),
        out_shape=jax.ShapeDtypeStruct((NVARS, 32), jnp.float32),
    )(ag0, ag1, cg, ah0, ah1, ch, x13v, gw2t, gb2, hw2t, hb2, w1a, w1b, w1c,
      b1, w2t, b2, gmidt)


def _ccuts_body(x_ref, w_ref, b_ref, out_ref):
    out_ref[...] = (jnp.dot(x_ref[...], w_ref[...],
                            preferred_element_type=jnp.float32) + b_ref[...])


def _tc_ccuts(x13c, wt, b):
    return pl.pallas_call(
        _ccuts_body,
        out_shape=jax.ShapeDtypeStruct((NCUTS, 32), jnp.float32),
    )(x13c, wt, b)


def _final_body(a0_ref, a1_ref, cc_ref, x_ref, gw2_ref, gb2_ref, w1a_ref,
                w1b_ref, b1_ref, w2_ref, b2_ref, out_ref):
    ac = _finalize_aggr(a0_ref[...], a1_ref[...], cc_ref[...], gw2_ref[...],
                        gb2_ref[...])
    h = jnp.maximum(
        jnp.dot(x_ref[...], w1a_ref[...], preferred_element_type=jnp.float32)
        + jnp.dot(ac, w1b_ref[...], preferred_element_type=jnp.float32)
        + b1_ref[...], 0.0)
    out_ref[...] = (jnp.dot(h, w2_ref[...], preferred_element_type=jnp.float32)
                    + b2_ref[...])


def _tc_final(a0, a1, cc, x13c, gw2t, gb2, w1a, w1b, b1, w2t, b2):
    return pl.pallas_call(
        _final_body,
        compiler_params=pltpu.CompilerParams(allow_input_fusion=[True] * 11),
        out_shape=jax.ShapeDtypeStruct((NDST_C, 32), jnp.float32),
    )(a0, a1, cc, x13c, gw2t, gb2, w1a, w1b, b1, w2t, b2)


def kernel(x_s, edge_index_s, edge_attr_s, vars_nodes, cuts_nodes, cons_edges,
           cuts_edges, gv_W1, gv_b1, gv_W2, gv_b2, hv_W1, hv_b1, hv_W2, hv_b2,
           fv_W1, fv_b1, fv_W2, fv_b2, gc_W1, gc_b1, gc_W2, gc_b2, fc_W1,
           fc_b1, fc_W2, fc_b2):
    f32 = jnp.float32
    i32 = jnp.int32
    rows = edge_index_s[0]
    cols = edge_index_s[1]
    attr = edge_attr_s[:, 0]
    x13 = x_s[:, :VF]
    x13c = x_s[NVARS + NCONS:, :VF]

    tg, th = _tc_proj(
        x13,
        gv_W1[:, :VF].T, gv_W1[:, VF:VF + CF].T, gv_b1[None, :],
        hv_W1[:, :VF].T, hv_W1[:, VF:VF + KF].T, hv_b1[None, :])
    tcc = _tc_ccuts(x13c, gc_W1[:, :KF].T, gc_b1[None, :])

    def pad(a, n, val):
        return jnp.concatenate([a, jnp.full((n - a.shape[0],), val, a.dtype)])

    def pack2(a, b, pe):
        return jnp.stack([a, b], 1).reshape(pe // CHUNK, CHUNK, 2
                                            ).transpose(0, 2, 1)

    zeros32v = jnp.zeros((NDST_V, 32), f32)
    zerocv = jnp.zeros((NDST_V,), f32)

    ie_g = pack2(pad(cols[:EC], PE_C, 0), pad(rows[:EC], PE_C, 0), PE_C)
    il_g = pack2(pad(cols[:EC].astype(f32), PE_C, float(NDST_V - 1)),
                 pad(attr[:EC], PE_C, 0.0), PE_C)
    acc_g, cnt_g = _edge_aggr_g(tg, ie_g, il_g, zeros32v, zerocv,
                                gv_W1[:, VF + CF])

    ie_h = pack2(pad(cols[EC:], PE_K, 0), pad(rows[EC:], PE_K, 0), PE_K)
    il_h = pack2(pad(cols[EC:].astype(f32), PE_K, float(NDST_V - 1)),
                 pad(attr[EC:], PE_K, 0.0), PE_K)
    acc_h, cnt_h = _edge_aggr_h(th, ie_h, il_h, zeros32v, zerocv,
                                hv_W1[:, VF + KF])

    tcv = _tc_cvars(
        acc_g[:NDST_V][:NVARS], acc_g[NDST_V:][:NVARS], cnt_g[:, :NVARS],
        acc_h[:NDST_V][:NVARS], acc_h[NDST_V:][:NVARS], cnt_h[:, :NVARS],
        x13[:NVARS],
        gv_W2.T, gv_b2[None, :], hv_W2.T, hv_b2[None, :],
        fv_W1[:, :VF].T, fv_W1[:, VF:VF + OC].T, fv_W1[:, VF + OC:].T,
        fv_b1[None, :], fv_W2.T, fv_b2[None, :],
        gc_W1[:, KF:KF + OC].T)
    tc_tab = jnp.concatenate([tcv, jnp.zeros((NCONS, 32), f32), tcc])

    ie_d = pack2(pad(rows[EC:], PE_K, 0), pad(cols[EC:], PE_K, 0), PE_K)
    il_d = pack2(pad((rows[EC:] - (NVARS + NCONS)).astype(f32), PE_K,
                     float(NDST_C - 1)),
                 pad(attr[EC:], PE_K, 0.0), PE_K)
    acc_c, cnt_c = _edge_aggr_d(tc_tab, ie_d, il_d,
                                jnp.zeros((NDST_C, 32), f32),
                                jnp.zeros((NDST_C,), f32), gc_W1[:, KF + OC])

    x13c_pad = jnp.concatenate([x13c, jnp.zeros((NDST_C - NCUTS, VF), f32)])
    out = _tc_final(
        acc_c[:NDST_C], acc_c[NDST_C:], cnt_c,
        x13c_pad, gc_W2.T, gc_b2[None, :],
        fc_W1[:, :VF].T, fc_W1[:, VF:].T, fc_b1[None, :],
        fc_W2.T, fc_b2[None, :])
    return out[:NCUTS]
